# baseline (speedup 1.0000x reference)
import functools

import jax
import jax.numpy as jnp
import numpy as np
from jax.experimental import pallas as pl
from jax.experimental.pallas import tpu as pltpu

EPS = 1e-5
SENS_MEAN = 0.00014507418272432547
LANES = 128


def _cdiv(a, b):
    return (a + b - 1) // b


WU = 64


def _apass_acc(rp_ref, su_ref, cu_ref, tbl_ref, acc_ref, g_scr):
    b = pl.program_id(0)
    lo = rp_ref[b]
    hi = rp_ref[b + 1]
    acc_ref[...] = jnp.zeros_like(acc_ref)
    a0 = lo // LANES
    nmic = (hi - a0 * LANES + LANES - 1) // LANES

    iota_sub = jax.lax.broadcasted_iota(jnp.int32, (WU, LANES), 0)
    iota_lane = jax.lax.broadcasted_iota(jnp.int32, (1, LANES), 1)

    def body(m, _):
        r = a0 + m
        suv = su_ref[r, 0, :].reshape(1, LANES)
        pos = r * LANES + iota_lane
        valid = (pos >= lo) & (pos < hi)
        local = suv - b * WU
        oh = jnp.where((iota_sub == local) & valid, 1.0, 0.0)
        for mi in range(LANES):
            idx = cu_ref[r, 0, mi]
            g_scr[mi, :] = tbl_ref[idx, 0, :]
        acc_ref[...] += jnp.dot(oh, g_scr[...],
                                preferred_element_type=jnp.float32)
        return 0

    jax.lax.fori_loop(0, nmic, body, 0)


def _a12_kernel(rp_ref, su_ref, cu_ref, tbl_ref, sp_ref,
                u2_ref, s01_ref, ut2_ref, acc_ref, g_scr):
    _apass_acc(rp_ref, su_ref, cu_ref, tbl_ref, acc_ref, g_scr)
    acc = acc_ref[...]
    a_e0 = acc[:, 0:LANES]
    a_i1 = acc[:, LANES:2 * LANES]
    sp = sp_ref[...]
    du = sp[:, 68:69]
    sqdu = sp[:, 69:70]
    u1 = sqdu * a_e0 + du * sp
    u2 = sqdu * a_i1 + du * u1
    lane = jax.lax.broadcasted_iota(jnp.int32, (WU, LANES), 1)
    feat = lane < 68
    u2_ref[...] = u2
    s01_ref[...] = jnp.where(feat, sp + u1 + u2, sp)
    ut2_ref[...] = jnp.where(feat, sqdu * u2, 0.0)


def _a34_kernel(rp_ref, su_ref, cu_ref, tbl_ref, s01_ref, u2_ref,
                emb_ref, ut3_ref, msum_ref, acc_ref, g_scr):
    _apass_acc(rp_ref, su_ref, cu_ref, tbl_ref, acc_ref, g_scr)
    acc = acc_ref[...]
    a_i2 = acc[:, 0:LANES]
    a_i3 = acc[:, LANES:2 * LANES]
    s01 = s01_ref[...]
    u2 = u2_ref[...]
    du = s01[:, 68:69]
    sqdu = s01[:, 69:70]
    u3 = sqdu * a_i2 + du * u2
    u4 = sqdu * a_i3 + du * u3
    lane = jax.lax.broadcasted_iota(jnp.int32, (WU, LANES), 1)
    feat = lane < 68
    emb = jnp.where(feat, s01 + u3 + u4, s01)
    emb_ref[...] = emb
    ut3_ref[...] = jnp.where(feat, sqdu * u3, 0.0)
    msum_ref[...] = jnp.sum(jnp.where(feat, emb, 0.0), axis=0).reshape(1, 1, LANES)


def _tpass_kernel(rp_ref, ki_ref, si_ref, stbl_ref, out_ref, acc_ref, g_scr,
                  *, ep, nbq, qsize, we):
    q = pl.program_id(0)
    wb = pl.program_id(1)
    blk = q * nbq + wb
    lo = rp_ref[blk]
    hi = rp_ref[blk + 1]
    acc_ref[...] = jnp.zeros_like(acc_ref)
    a0 = lo // LANES
    nmic = (hi - a0 * LANES + LANES - 1) // LANES

    iota_sub = jax.lax.broadcasted_iota(jnp.int32, (we, LANES), 0)
    iota_lane = jax.lax.broadcasted_iota(jnp.int32, (1, LANES), 1)
    base = q * ep + wb * we
    qoff = q * qsize

    def body(m, _):
        r = a0 + m
        kiv = ki_ref[r, 0, :].reshape(1, LANES)
        pos = r * LANES + iota_lane
        valid = (pos >= lo) & (pos < hi)
        local = kiv - base
        oh = jnp.where((iota_sub == local) & valid, 1.0, 0.0)
        for mi in range(LANES):
            idx = si_ref[r, 0, mi] - qoff
            idx = jnp.clip(idx, 0, qsize - 1)
            g_scr[mi, :] = stbl_ref[idx, 0, :]
        acc_ref[...] += jnp.dot(oh, g_scr[...],
                                preferred_element_type=jnp.float32)
        return 0

    jax.lax.fori_loop(0, nmic, body, 0)
    out_ref[...] = acc_ref[...].reshape(1, we, LANES)


def _graph_embeddings(edge_stu, edge_exer, s0, e0, alpha, beta, d_u, d_i):
    S, K = s0.shape
    E = e0.shape[0]
    NE = edge_stu.shape[0]

    NBU = _cdiv(S, 4 * WU) * 4
    SP = NBU * WU
    QS = SP // 4
    WE = 64
    NBQ = _cdiv(E, WE)
    EP = NBQ * WE
    NBI = 4 * NBQ
    NR = _cdiv(NE, LANES)
    NEP = NR * LANES

    f32 = jnp.float32
    du = 1.0 / (d_u + 1.0)
    di = 1.0 / (d_i + 1.0)
    sqdu = jnp.sqrt(du)
    sqdi = jnp.sqrt(di)

    su, cu = jax.lax.sort((edge_stu, edge_exer), num_keys=1)
    qid = edge_stu // QS
    kI, sI = jax.lax.sort((qid * EP + edge_exer, edge_stu), num_keys=1)

    def pad_edges(a, sent):
        a = jnp.concatenate([a, jnp.full((NEP - NE,), sent, a.dtype)])
        return a.reshape(NR, 1, LANES)

    su3 = pad_edges(su, S)
    cu3 = pad_edges(cu, 0)
    kI3 = pad_edges(kI, 4 * EP)
    sI3 = pad_edges(sI, 0)

    degu_p = jnp.concatenate([d_u, jnp.zeros((SP - S,), f32)])
    cnt_u = degu_p.reshape(NBU, WU).sum(1)
    rpU = jnp.concatenate([jnp.zeros((1,), f32), jnp.cumsum(cnt_u)]).astype(jnp.int32)
    degq = jax.ops.segment_sum(jnp.ones((NE,), f32), qid * EP + edge_exer,
                               num_segments=4 * EP)
    cnt_i = degq.reshape(NBI, WE).sum(1)
    rpI = jnp.concatenate([jnp.zeros((1,), f32), jnp.cumsum(cnt_i)]).astype(jnp.int32)

    def padlane(x, rows, width=LANES):
        out = jnp.zeros((rows, width), f32)
        return out.at[:x.shape[0], :x.shape[1]].set(x)

    sp_pack = padlane(
        jnp.concatenate([s0, du[:, None], sqdu[:, None], alpha, beta], axis=1), SP)
    st0 = padlane(s0 * sqdu[:, None], SP).reshape(SP, 1, LANES)

    e0p = padlane(e0, EP)
    sqdi_p = jnp.concatenate([sqdi, jnp.zeros((EP - E,), f32)])[:, None]
    di_p = jnp.concatenate([di, jnp.zeros((EP - E,), f32)])[:, None]

    tp = functools.partial(_tpass_kernel, ep=EP, nbq=NBQ, qsize=QS, we=WE)
    tpass = pl.pallas_call(
        tp,
        grid_spec=pltpu.PrefetchScalarGridSpec(
            num_scalar_prefetch=1,
            grid=(4, NBQ),
            in_specs=[
                pl.BlockSpec((NR, 1, LANES), lambda q, w, rp: (0, 0, 0)),
                pl.BlockSpec((NR, 1, LANES), lambda q, w, rp: (0, 0, 0)),
                pl.BlockSpec((QS, 1, LANES), lambda q, w, rp: (q, 0, 0)),
            ],
            out_specs=pl.BlockSpec((1, WE, LANES), lambda q, w, rp: (q * NBQ + w, 0, 0)),
            scratch_shapes=[
                pltpu.VMEM((WE, LANES), f32),
                pltpu.VMEM((LANES, LANES), f32),
            ],
        ),
        out_shape=jax.ShapeDtypeStruct((NBI, WE, LANES), f32),
        compiler_params=pltpu.CompilerParams(
            dimension_semantics=("parallel", "arbitrary")),
        name="tpass",
    )

    def run_t(stbl):
        acc = tpass(rpI, kI3, sI3, stbl)
        return acc.reshape(4, NBQ * WE, LANES).sum(0)

    bt_s0 = run_t(st0)
    i1 = sqdi_p * bt_s0 + di_p * e0p

    tblA12 = jnp.concatenate([sqdi_p * e0p, sqdi_p * i1], axis=1).reshape(EP, 1, 2 * LANES)

    apass12 = pl.pallas_call(
        _a12_kernel,
        grid_spec=pltpu.PrefetchScalarGridSpec(
            num_scalar_prefetch=1,
            grid=(NBU,),
            in_specs=[
                pl.BlockSpec((NR, 1, LANES), lambda b, rp: (0, 0, 0)),
                pl.BlockSpec((NR, 1, LANES), lambda b, rp: (0, 0, 0)),
                pl.BlockSpec((EP, 1, 2 * LANES), lambda b, rp: (0, 0, 0)),
                pl.BlockSpec((WU, LANES), lambda b, rp: (b, 0)),
            ],
            out_specs=[
                pl.BlockSpec((WU, LANES), lambda b, rp: (b, 0)),
                pl.BlockSpec((WU, LANES), lambda b, rp: (b, 0)),
                pl.BlockSpec((WU, LANES), lambda b, rp: (b, 0)),
            ],
            scratch_shapes=[
                pltpu.VMEM((WU, 2 * LANES), f32),
                pltpu.VMEM((LANES, 2 * LANES), f32),
            ],
        ),
        out_shape=[
            jax.ShapeDtypeStruct((SP, LANES), f32),
            jax.ShapeDtypeStruct((SP, LANES), f32),
            jax.ShapeDtypeStruct((SP, LANES), f32),
        ],
        compiler_params=pltpu.CompilerParams(dimension_semantics=("parallel",)),
        name="apass12",
    )
    u2a, s01a, ut2a = apass12(rpU, su3, cu3, tblA12, sp_pack)

    bt_u2 = run_t(ut2a.reshape(SP, 1, LANES))
    i2 = sqdi_p * bt_u2 + di_p * i1
    i3 = i2 + di_p * (i2 - i1)

    tblA34 = jnp.concatenate([sqdi_p * i2, sqdi_p * i3], axis=1).reshape(EP, 1, 2 * LANES)

    apass34 = pl.pallas_call(
        _a34_kernel,
        grid_spec=pltpu.PrefetchScalarGridSpec(
            num_scalar_prefetch=1,
            grid=(NBU,),
            in_specs=[
                pl.BlockSpec((NR, 1, LANES), lambda b, rp: (0, 0, 0)),
                pl.BlockSpec((NR, 1, LANES), lambda b, rp: (0, 0, 0)),
                pl.BlockSpec((EP, 1, 2 * LANES), lambda b, rp: (0, 0, 0)),
                pl.BlockSpec((WU, LANES), lambda b, rp: (b, 0)),
                pl.BlockSpec((WU, LANES), lambda b, rp: (b, 0)),
            ],
            out_specs=[
                pl.BlockSpec((WU, LANES), lambda b, rp: (b, 0)),
                pl.BlockSpec((WU, LANES), lambda b, rp: (b, 0)),
                pl.BlockSpec((1, 1, LANES), lambda b, rp: (b, 0, 0)),
            ],
            scratch_shapes=[
                pltpu.VMEM((WU, 2 * LANES), f32),
                pltpu.VMEM((LANES, 2 * LANES), f32),
            ],
        ),
        out_shape=[
            jax.ShapeDtypeStruct((SP, LANES), f32),
            jax.ShapeDtypeStruct((SP, LANES), f32),
            jax.ShapeDtypeStruct((NBU, 1, LANES), f32),
        ],
        compiler_params=pltpu.CompilerParams(dimension_semantics=("parallel",)),
        name="apass34",
    )
    emb, ut3a, msum = apass34(rpU, su3, cu3, tblA34, s01a, u2a)

    bt_u3 = run_t(ut3a.reshape(SP, 1, LANES))
    i4 = sqdi_p * bt_u3 + di_p * i3

    exer_emb = e0p + i1 + i2 + i3 + i4
    mean_stu = msum.sum(axis=(0, 1))[:K] / S
    return emb, mean_stu, exer_emb


def _matT(x, w):
    return jax.lax.dot_general(x, w, (((1,), (1,)), ((), ())),
                               preferred_element_type=jnp.float32)


def _lin_r(x, wr, br):
    return _matT(x, wr[...]) + br[...]


def _bn_r(x, gr, sr):
    m = jnp.mean(x, axis=0, keepdims=True)
    v = jnp.mean(jnp.square(x - m), axis=0, keepdims=True)
    return gr[...] * (x - m) * jax.lax.rsqrt(v + EPS) + sr[...]


def _mlp3_r(x, p):
    h = jax.nn.relu(_bn_r(_lin_r(x, p[0], p[1]), p[2], p[3]))
    h = jax.nn.relu(_bn_r(_lin_r(h, p[4], p[5]), p[6], p[7]))
    return _bn_r(_lin_r(h, p[8], p[9]), p[10], p[11])


def _rev_r(x, p):
    h = jax.nn.relu(_bn_r(_lin_r(x, p[0], p[1]), p[2], p[3]))
    return _bn_r(_lin_r(h, p[4], p[5]), p[6], p[7])[:, 0:1]


def _pos_lin_r(x, wr, br):
    w = wr[...]
    return _matT(x, 2.0 * jax.nn.relu(-w) + w) + br[...]


def _dense_kernel(stu_ref, ex_ref, sens_ref, kp_ref, clsl_ref, mean_ref,
                  *refs):
    out_ref, rev_ref, cls_ref = refs[-3:]
    w = list(refs[:-3])
    K = 68
    B = stu_ref.shape[0]
    p_comb, w = w[:12], w[12:]
    p_sens, w = w[:12], w[12:]
    p_sdense, w = w[:12], w[12:]
    p_rev, w = w[:8], w[8:]
    p_cls = [None] * 5
    for i in range(5):
        p_cls[i], w = w[:6], w[6:]
    (pred1w, pred1b, pred2w, pred2b, pred3w, pred3b) = w

    stu = stu_ref[...]
    stu_feat = stu[:, :K]
    alpha = jax.nn.sigmoid(stu[:, 70:71])
    beta = jax.nn.sigmoid(stu[:, 71:72])
    ex = ex_ref[...]
    k_diff = jax.nn.sigmoid(ex[:, :K])
    e_diff = jax.nn.sigmoid(ex[:, K:K + 1])
    sens8 = sens_ref[...]
    sens = sens8[:, 0:1]

    kp = kp_ref[...]
    iota_k = jax.lax.broadcasted_iota(jnp.int32, (B, K), 1)
    mh = jnp.zeros((B, K), jnp.float32)
    for j in range(4):
        mh = jnp.maximum(mh, jnp.where(iota_k == kp[:, j:j + 1], 1.0, 0.0))

    sens_feat = _mlp3_r(sens8, p_sens)
    Uf_features = _mlp3_r(jnp.concatenate([stu_feat, sens_feat], axis=-1),
                          p_comb)
    Ud_features = _mlp3_r(sens_feat, p_sdense)
    Uf_rev = _rev_r(Uf_features, p_rev)
    Ud_rev = _rev_r(Ud_features, p_rev)

    reverse_loss = (jnp.mean(jnp.square(Uf_rev - SENS_MEAN))
                    + jnp.mean(jnp.square(Ud_rev - sens)))
    rev_ref[...] = reverse_loss.reshape(1, 1)

    Uf = jax.nn.sigmoid(Uf_features)
    Ud = jax.nn.sigmoid(Ud_features)
    stat_emb = jax.nn.sigmoid((1.0 - alpha) * Uf + alpha * Ud)

    con_stu = jnp.broadcast_to(mean_ref[...], (B, K))
    con_col = jnp.full((B, 8), SENS_MEAN, jnp.float32)
    con_sens = _mlp3_r(con_col, p_sens)
    con_Uf = jax.nn.sigmoid(
        _mlp3_r(jnp.concatenate([con_stu, con_sens], axis=-1), p_comb))
    con_stat = jax.nn.sigmoid((1.0 - alpha) * con_Uf + alpha * Ud)

    clsl = clsl_ref[...]
    cls_loss = 0.0
    for i in range(5):
        pc = p_cls[i]
        h = jax.nn.relu(_bn_r(_lin_r(Uf_features, pc[0], pc[1]), pc[2], pc[3]))
        oi = jax.nn.sigmoid(_lin_r(h, pc[4], pc[5])[:, 0:1])
        y = clsl[:, i:i + 1]
        cls_loss = cls_loss + jnp.mean(jax.nn.softplus(oi) - oi * y)
    cls_ref[...] = (cls_loss / 5.0).reshape(1, 1)

    debias_theta = jax.nn.sigmoid(stat_emb - beta * con_stat)
    x = e_diff * (debias_theta - k_diff) * mh
    x = jax.nn.sigmoid(_pos_lin_r(x, pred1w, pred1b))
    x = jax.nn.sigmoid(_pos_lin_r(x, pred2w, pred2b))
    out_ref[...] = jax.nn.sigmoid(_pos_lin_r(x, pred3w, pred3b)[:, 0:1])


def _dense_head(stu_rows, ex_rows, sensitive, kp, cls_labels, mean_stu, params):
    B = stu_rows.shape[0]
    f32 = jnp.float32

    def pad8(a):
        if a.shape[0] == 1 and a.shape[-1] > 1:
            a = jnp.pad(a, ((0, 7), (0, 0)))
        if a.shape[-1] == 1 and a.shape[0] > 1:
            a = jnp.pad(a, ((0, 0), (0, 7)))
        return a

    def padb(a):
        if a.shape[-1] == 1:
            return jnp.pad(a, ((0, 0), (0, 7)))
        return a

    def blk(p, n):
        out = []
        for i in range(n):
            j = i + 1
            out += [pad8(p['W%d' % j]), padb(p['b%d' % j].reshape(1, -1)),
                    padb(p['g%d' % j].reshape(1, -1)),
                    padb(p['s%d' % j].reshape(1, -1))]
        return out

    wlist = blk(params['combine'], 3) + blk(params['sens'], 3) \
        + blk(params['sens_dense'], 3) + blk(params['sens_rev'], 2)
    for i in range(5):
        p = params['cls'][i]
        wlist += [pad8(p['W1']), padb(p['b1'].reshape(1, -1)),
                  padb(p['g1'].reshape(1, -1)), padb(p['s1'].reshape(1, -1)),
                  pad8(p['W2']), padb(p['b2'].reshape(1, -1))]
    for nm in ('pred1', 'pred2', 'pred3'):
        wlist += [pad8(params[nm]['W']), padb(params[nm]['b'].reshape(1, -1))]

    args = [stu_rows, ex_rows,
            jnp.pad(sensitive.reshape(B, 1), ((0, 0), (0, 7))), kp,
            cls_labels.astype(f32).T, mean_stu.reshape(1, -1)] + wlist
    out, rev, cls = pl.pallas_call(
        _dense_kernel,
        out_shape=[
            jax.ShapeDtypeStruct((B, 1), f32),
            jax.ShapeDtypeStruct((1, 1), f32),
            jax.ShapeDtypeStruct((1, 1), f32),
        ],
        compiler_params=pltpu.CompilerParams(
            vmem_limit_bytes=63 * 1024 * 1024),
        name="dense_head",
    )(*args)
    return out, rev[0, 0], cls[0, 0]


def kernel(stu_id, input_exercise, input_knowledge_point, sensitive, labels,
           cls_labels, edge_stu, edge_exer, params):
    S, K = params['student_emb'].shape
    E = params['k_diff'].shape[0]
    NE = edge_stu.shape[0]
    B = stu_id.shape[0]
    f32 = jnp.float32

    ones = jnp.ones((NE,), f32)
    d_u = jax.ops.segment_sum(ones, edge_stu, num_segments=S)
    d_i = jax.ops.segment_sum(ones, edge_exer, num_segments=E)

    emb, mean_stu, exer_emb = _graph_embeddings(
        edge_stu, edge_exer, params['student_emb'], params['k_diff'],
        params['alpha'], params['beta'], d_u, d_i)

    exer_pack = jax.lax.dynamic_update_slice(
        exer_emb, params['e_diff'], (0, K))

    stu_rows = emb[stu_id]
    ex_rows = exer_pack[input_exercise]

    output, reverse_loss, cls_loss = _dense_head(
        stu_rows, ex_rows, sensitive, input_knowledge_point, cls_labels,
        mean_stu, params)
    return output, reverse_loss, cls_loss


# speedup vs baseline: 1.2737x; 1.2737x over previous
import functools

import jax
import jax.numpy as jnp
import numpy as np
from jax.experimental import pallas as pl
from jax.experimental.pallas import tpu as pltpu

EPS = 1e-5
SENS_MEAN = 0.00014507418272432547
LANES = 128


def _cdiv(a, b):
    return (a + b - 1) // b


WU = 128


SUBR = 8


def _apass_acc(rp_ref, su_ref, cu_ref, tbl_ref, acc_ref, g_scr, oh_scr):
    b = pl.program_id(0)
    lo = rp_ref[b]
    hi = rp_ref[b + 1]
    acc_ref[...] = jnp.zeros_like(acc_ref)
    a0 = lo // LANES
    nch = (hi - a0 * LANES + SUBR * LANES - 1) // (SUBR * LANES)

    iota_sub = jax.lax.broadcasted_iota(jnp.int32, (WU, LANES), 0)
    iota_lane = jax.lax.broadcasted_iota(jnp.int32, (1, LANES), 1)

    def body(ch, _):
        r0 = a0 + ch * SUBR
        for sm in range(SUBR):
            r = r0 + sm
            suv = su_ref[r, 0, :].reshape(1, LANES)
            pos = r * LANES + iota_lane
            valid = (pos >= lo) & (pos < hi)
            local = suv - b * WU
            oh_scr[:, sm * LANES:(sm + 1) * LANES] = jnp.where(
                (iota_sub == local) & valid, 1.0, 0.0)
            for mi in range(LANES):
                idx = cu_ref[r, 0, mi]
                g_scr[sm * LANES + mi, :] = tbl_ref[idx, 0, :]
        acc_ref[...] += jnp.dot(oh_scr[...], g_scr[...],
                                preferred_element_type=jnp.float32)
        return 0

    jax.lax.fori_loop(0, nch, body, 0)


def _a12_kernel(rp_ref, su_ref, cu_ref, tbl_ref, sp_ref,
                u2_ref, s01_ref, ut2_ref, acc_ref, g_scr, oh_scr):
    _apass_acc(rp_ref, su_ref, cu_ref, tbl_ref, acc_ref, g_scr, oh_scr)
    acc = acc_ref[...]
    a_e0 = acc[:, 0:LANES]
    a_i1 = acc[:, LANES:2 * LANES]
    sp = sp_ref[...]
    du = sp[:, 68:69]
    sqdu = sp[:, 69:70]
    u1 = sqdu * a_e0 + du * sp
    u2 = sqdu * a_i1 + du * u1
    lane = jax.lax.broadcasted_iota(jnp.int32, (WU, LANES), 1)
    feat = lane < 68
    u2_ref[...] = u2
    s01_ref[...] = jnp.where(feat, sp + u1 + u2, sp)
    ut2_ref[...] = jnp.where(feat, sqdu * u2, 0.0)


def _a34_kernel(rp_ref, su_ref, cu_ref, tbl_ref, s01_ref, u2_ref,
                emb_ref, ut3_ref, msum_ref, acc_ref, g_scr, oh_scr):
    _apass_acc(rp_ref, su_ref, cu_ref, tbl_ref, acc_ref, g_scr, oh_scr)
    acc = acc_ref[...]
    a_i2 = acc[:, 0:LANES]
    a_i3 = acc[:, LANES:2 * LANES]
    s01 = s01_ref[...]
    u2 = u2_ref[...]
    du = s01[:, 68:69]
    sqdu = s01[:, 69:70]
    u3 = sqdu * a_i2 + du * u2
    u4 = sqdu * a_i3 + du * u3
    lane = jax.lax.broadcasted_iota(jnp.int32, (WU, LANES), 1)
    feat = lane < 68
    emb = jnp.where(feat, s01 + u3 + u4, s01)
    emb_ref[...] = emb
    ut3_ref[...] = jnp.where(feat, sqdu * u3, 0.0)
    msum_ref[...] = jnp.sum(jnp.where(feat, emb, 0.0), axis=0).reshape(1, 1, LANES)


def _tpass_kernel(rp_ref, ki_ref, si_ref, stbl_ref, out_ref, acc_ref, g_scr,
                  oh_scr, *, ep, nbq, qsize, we):
    q = pl.program_id(0)
    wb = pl.program_id(1)
    blk = q * nbq + wb
    lo = rp_ref[blk]
    hi = rp_ref[blk + 1]
    acc_ref[...] = jnp.zeros_like(acc_ref)
    a0 = lo // LANES
    nch = (hi - a0 * LANES + SUBR * LANES - 1) // (SUBR * LANES)

    iota_sub = jax.lax.broadcasted_iota(jnp.int32, (we, LANES), 0)
    iota_lane = jax.lax.broadcasted_iota(jnp.int32, (1, LANES), 1)
    base = q * ep + wb * we
    qoff = q * qsize

    def body(ch, _):
        r0 = a0 + ch * SUBR
        for sm in range(SUBR):
            r = r0 + sm
            kiv = ki_ref[r, 0, :].reshape(1, LANES)
            pos = r * LANES + iota_lane
            valid = (pos >= lo) & (pos < hi)
            local = kiv - base
            oh_scr[:, sm * LANES:(sm + 1) * LANES] = jnp.where(
                (iota_sub == local) & valid, 1.0, 0.0)
            for mi in range(LANES):
                idx = si_ref[r, 0, mi] - qoff
                idx = jnp.clip(idx, 0, qsize - 1)
                g_scr[sm * LANES + mi, :] = stbl_ref[idx, 0, :]
        acc_ref[...] += jnp.dot(oh_scr[...], g_scr[...],
                                preferred_element_type=jnp.float32)
        return 0

    jax.lax.fori_loop(0, nch, body, 0)
    out_ref[...] = acc_ref[...].reshape(1, we, LANES)


def _graph_embeddings(edge_stu, edge_exer, s0, e0, alpha, beta, d_u, d_i):
    S, K = s0.shape
    E = e0.shape[0]
    NE = edge_stu.shape[0]

    NBU = _cdiv(S, 4 * WU) * 4
    SP = NBU * WU
    QS = SP // 4
    WE = 64
    NBQ = _cdiv(E, WE)
    EP = NBQ * WE
    NBI = 4 * NBQ
    NR = _cdiv(NE, LANES) + 2 * SUBR
    NEP = NR * LANES

    f32 = jnp.float32
    du = 1.0 / (d_u + 1.0)
    di = 1.0 / (d_i + 1.0)
    sqdu = jnp.sqrt(du)
    sqdi = jnp.sqrt(di)

    su, cu = jax.lax.sort((edge_stu, edge_exer), num_keys=1)
    qid = edge_stu // QS
    kI, sI = jax.lax.sort((qid * EP + edge_exer, edge_stu), num_keys=1)

    def pad_edges(a, sent):
        a = jnp.concatenate([a, jnp.full((NEP - NE,), sent, a.dtype)])
        return a.reshape(NR, 1, LANES)

    su3 = pad_edges(su, S)
    cu3 = pad_edges(cu, 0)
    kI3 = pad_edges(kI, 4 * EP)
    sI3 = pad_edges(sI, 0)

    degu_p = jnp.concatenate([d_u, jnp.zeros((SP - S,), f32)])
    cnt_u = degu_p.reshape(NBU, WU).sum(1)
    rpU = jnp.concatenate([jnp.zeros((1,), f32), jnp.cumsum(cnt_u)]).astype(jnp.int32)
    degq = jax.ops.segment_sum(jnp.ones((NE,), f32), qid * EP + edge_exer,
                               num_segments=4 * EP)
    cnt_i = degq.reshape(NBI, WE).sum(1)
    rpI = jnp.concatenate([jnp.zeros((1,), f32), jnp.cumsum(cnt_i)]).astype(jnp.int32)

    def padlane(x, rows, width=LANES):
        out = jnp.zeros((rows, width), f32)
        return out.at[:x.shape[0], :x.shape[1]].set(x)

    sp_pack = padlane(
        jnp.concatenate([s0, du[:, None], sqdu[:, None], alpha, beta], axis=1), SP)
    st0 = padlane(s0 * sqdu[:, None], SP).reshape(SP, 1, LANES)

    e0p = padlane(e0, EP)
    sqdi_p = jnp.concatenate([sqdi, jnp.zeros((EP - E,), f32)])[:, None]
    di_p = jnp.concatenate([di, jnp.zeros((EP - E,), f32)])[:, None]

    tp = functools.partial(_tpass_kernel, ep=EP, nbq=NBQ, qsize=QS, we=WE)
    tpass = pl.pallas_call(
        tp,
        grid_spec=pltpu.PrefetchScalarGridSpec(
            num_scalar_prefetch=1,
            grid=(4, NBQ),
            in_specs=[
                pl.BlockSpec((NR, 1, LANES), lambda q, w, rp: (0, 0, 0)),
                pl.BlockSpec((NR, 1, LANES), lambda q, w, rp: (0, 0, 0)),
                pl.BlockSpec((QS, 1, LANES), lambda q, w, rp: (q, 0, 0)),
            ],
            out_specs=pl.BlockSpec((1, WE, LANES), lambda q, w, rp: (q * NBQ + w, 0, 0)),
            scratch_shapes=[
                pltpu.VMEM((WE, LANES), f32),
                pltpu.VMEM((8 * LANES, LANES), f32),
                pltpu.VMEM((WE, 8 * LANES), f32),
            ],
        ),
        out_shape=jax.ShapeDtypeStruct((NBI, WE, LANES), f32),
        compiler_params=pltpu.CompilerParams(
            dimension_semantics=("parallel", "arbitrary")),
        name="tpass",
    )

    def run_t(stbl):
        acc = tpass(rpI, kI3, sI3, stbl)
        return acc.reshape(4, NBQ * WE, LANES).sum(0)

    bt_s0 = run_t(st0)
    i1 = sqdi_p * bt_s0 + di_p * e0p

    tblA12 = jnp.concatenate([sqdi_p * e0p, sqdi_p * i1], axis=1).reshape(EP, 1, 2 * LANES)

    apass12 = pl.pallas_call(
        _a12_kernel,
        grid_spec=pltpu.PrefetchScalarGridSpec(
            num_scalar_prefetch=1,
            grid=(NBU,),
            in_specs=[
                pl.BlockSpec((NR, 1, LANES), lambda b, rp: (0, 0, 0)),
                pl.BlockSpec((NR, 1, LANES), lambda b, rp: (0, 0, 0)),
                pl.BlockSpec((EP, 1, 2 * LANES), lambda b, rp: (0, 0, 0)),
                pl.BlockSpec((WU, LANES), lambda b, rp: (b, 0)),
            ],
            out_specs=[
                pl.BlockSpec((WU, LANES), lambda b, rp: (b, 0)),
                pl.BlockSpec((WU, LANES), lambda b, rp: (b, 0)),
                pl.BlockSpec((WU, LANES), lambda b, rp: (b, 0)),
            ],
            scratch_shapes=[
                pltpu.VMEM((WU, 2 * LANES), f32),
                pltpu.VMEM((8 * LANES, 2 * LANES), f32),
                pltpu.VMEM((WU, 8 * LANES), f32),
            ],
        ),
        out_shape=[
            jax.ShapeDtypeStruct((SP, LANES), f32),
            jax.ShapeDtypeStruct((SP, LANES), f32),
            jax.ShapeDtypeStruct((SP, LANES), f32),
        ],
        compiler_params=pltpu.CompilerParams(dimension_semantics=("parallel",)),
        name="apass12",
    )
    u2a, s01a, ut2a = apass12(rpU, su3, cu3, tblA12, sp_pack)

    bt_u2 = run_t(ut2a.reshape(SP, 1, LANES))
    i2 = sqdi_p * bt_u2 + di_p * i1
    i3 = i2 + di_p * (i2 - i1)

    tblA34 = jnp.concatenate([sqdi_p * i2, sqdi_p * i3], axis=1).reshape(EP, 1, 2 * LANES)

    apass34 = pl.pallas_call(
        _a34_kernel,
        grid_spec=pltpu.PrefetchScalarGridSpec(
            num_scalar_prefetch=1,
            grid=(NBU,),
            in_specs=[
                pl.BlockSpec((NR, 1, LANES), lambda b, rp: (0, 0, 0)),
                pl.BlockSpec((NR, 1, LANES), lambda b, rp: (0, 0, 0)),
                pl.BlockSpec((EP, 1, 2 * LANES), lambda b, rp: (0, 0, 0)),
                pl.BlockSpec((WU, LANES), lambda b, rp: (b, 0)),
                pl.BlockSpec((WU, LANES), lambda b, rp: (b, 0)),
            ],
            out_specs=[
                pl.BlockSpec((WU, LANES), lambda b, rp: (b, 0)),
                pl.BlockSpec((WU, LANES), lambda b, rp: (b, 0)),
                pl.BlockSpec((1, 1, LANES), lambda b, rp: (b, 0, 0)),
            ],
            scratch_shapes=[
                pltpu.VMEM((WU, 2 * LANES), f32),
                pltpu.VMEM((8 * LANES, 2 * LANES), f32),
                pltpu.VMEM((WU, 8 * LANES), f32),
            ],
        ),
        out_shape=[
            jax.ShapeDtypeStruct((SP, LANES), f32),
            jax.ShapeDtypeStruct((SP, LANES), f32),
            jax.ShapeDtypeStruct((NBU, 1, LANES), f32),
        ],
        compiler_params=pltpu.CompilerParams(dimension_semantics=("parallel",)),
        name="apass34",
    )
    emb, ut3a, msum = apass34(rpU, su3, cu3, tblA34, s01a, u2a)

    bt_u3 = run_t(ut3a.reshape(SP, 1, LANES))
    i4 = sqdi_p * bt_u3 + di_p * i3

    exer_emb = e0p + i1 + i2 + i3 + i4
    mean_stu = msum.sum(axis=(0, 1))[:K] / S
    return emb, mean_stu, exer_emb


def _matT(x, w):
    return jax.lax.dot_general(x, w, (((1,), (1,)), ((), ())),
                               preferred_element_type=jnp.float32)


def _lin_r(x, wr, br):
    return _matT(x, wr[...]) + br[...]


def _bn_r(x, gr, sr):
    m = jnp.mean(x, axis=0, keepdims=True)
    v = jnp.mean(jnp.square(x - m), axis=0, keepdims=True)
    return gr[...] * (x - m) * jax.lax.rsqrt(v + EPS) + sr[...]


def _mlp3_r(x, p):
    h = jax.nn.relu(_bn_r(_lin_r(x, p[0], p[1]), p[2], p[3]))
    h = jax.nn.relu(_bn_r(_lin_r(h, p[4], p[5]), p[6], p[7]))
    return _bn_r(_lin_r(h, p[8], p[9]), p[10], p[11])


def _rev_r(x, p):
    h = jax.nn.relu(_bn_r(_lin_r(x, p[0], p[1]), p[2], p[3]))
    return _bn_r(_lin_r(h, p[4], p[5]), p[6], p[7])[:, 0:1]


def _pos_lin_r(x, wr, br):
    w = wr[...]
    return _matT(x, 2.0 * jax.nn.relu(-w) + w) + br[...]


def _dense_kernel(stu_ref, ex_ref, sens_ref, kp_ref, clsl_ref, mean_ref,
                  *refs):
    out_ref, rev_ref, cls_ref = refs[-3:]
    w = list(refs[:-3])
    K = 68
    B = stu_ref.shape[0]
    p_comb, w = w[:12], w[12:]
    p_sens, w = w[:12], w[12:]
    p_sdense, w = w[:12], w[12:]
    p_rev, w = w[:8], w[8:]
    p_cls = [None] * 5
    for i in range(5):
        p_cls[i], w = w[:6], w[6:]
    (pred1w, pred1b, pred2w, pred2b, pred3w, pred3b) = w

    stu = stu_ref[...]
    stu_feat = stu[:, :K]
    alpha = jax.nn.sigmoid(stu[:, 70:71])
    beta = jax.nn.sigmoid(stu[:, 71:72])
    ex = ex_ref[...]
    k_diff = jax.nn.sigmoid(ex[:, :K])
    e_diff = jax.nn.sigmoid(ex[:, K:K + 1])
    sens8 = sens_ref[...]
    sens = sens8[:, 0:1]

    kp = kp_ref[...]
    iota_k = jax.lax.broadcasted_iota(jnp.int32, (B, K), 1)
    mh = jnp.zeros((B, K), jnp.float32)
    for j in range(4):
        mh = jnp.maximum(mh, jnp.where(iota_k == kp[:, j:j + 1], 1.0, 0.0))

    sens_feat = _mlp3_r(sens8, p_sens)
    Uf_features = _mlp3_r(jnp.concatenate([stu_feat, sens_feat], axis=-1),
                          p_comb)
    Ud_features = _mlp3_r(sens_feat, p_sdense)
    Uf_rev = _rev_r(Uf_features, p_rev)
    Ud_rev = _rev_r(Ud_features, p_rev)

    reverse_loss = (jnp.mean(jnp.square(Uf_rev - SENS_MEAN))
                    + jnp.mean(jnp.square(Ud_rev - sens)))
    rev_ref[...] = reverse_loss.reshape(1, 1)

    Uf = jax.nn.sigmoid(Uf_features)
    Ud = jax.nn.sigmoid(Ud_features)
    stat_emb = jax.nn.sigmoid((1.0 - alpha) * Uf + alpha * Ud)

    con_stu = jnp.broadcast_to(mean_ref[...], (B, K))
    con_col = jnp.full((B, 8), SENS_MEAN, jnp.float32)
    con_sens = _mlp3_r(con_col, p_sens)
    con_Uf = jax.nn.sigmoid(
        _mlp3_r(jnp.concatenate([con_stu, con_sens], axis=-1), p_comb))
    con_stat = jax.nn.sigmoid((1.0 - alpha) * con_Uf + alpha * Ud)

    clsl = clsl_ref[...]
    cls_loss = 0.0
    for i in range(5):
        pc = p_cls[i]
        h = jax.nn.relu(_bn_r(_lin_r(Uf_features, pc[0], pc[1]), pc[2], pc[3]))
        oi = jax.nn.sigmoid(_lin_r(h, pc[4], pc[5])[:, 0:1])
        y = clsl[:, i:i + 1]
        cls_loss = cls_loss + jnp.mean(jax.nn.softplus(oi) - oi * y)
    cls_ref[...] = (cls_loss / 5.0).reshape(1, 1)

    debias_theta = jax.nn.sigmoid(stat_emb - beta * con_stat)
    x = e_diff * (debias_theta - k_diff) * mh
    x = jax.nn.sigmoid(_pos_lin_r(x, pred1w, pred1b))
    x = jax.nn.sigmoid(_pos_lin_r(x, pred2w, pred2b))
    out_ref[...] = jax.nn.sigmoid(_pos_lin_r(x, pred3w, pred3b)[:, 0:1])


def _dense_head(stu_rows, ex_rows, sensitive, kp, cls_labels, mean_stu, params):
    B = stu_rows.shape[0]
    f32 = jnp.float32

    def pad8(a):
        if a.shape[0] == 1 and a.shape[-1] > 1:
            a = jnp.pad(a, ((0, 7), (0, 0)))
        if a.shape[-1] == 1 and a.shape[0] > 1:
            a = jnp.pad(a, ((0, 0), (0, 7)))
        return a

    def padb(a):
        if a.shape[-1] == 1:
            return jnp.pad(a, ((0, 0), (0, 7)))
        return a

    def blk(p, n):
        out = []
        for i in range(n):
            j = i + 1
            out += [pad8(p['W%d' % j]), padb(p['b%d' % j].reshape(1, -1)),
                    padb(p['g%d' % j].reshape(1, -1)),
                    padb(p['s%d' % j].reshape(1, -1))]
        return out

    wlist = blk(params['combine'], 3) + blk(params['sens'], 3) \
        + blk(params['sens_dense'], 3) + blk(params['sens_rev'], 2)
    for i in range(5):
        p = params['cls'][i]
        wlist += [pad8(p['W1']), padb(p['b1'].reshape(1, -1)),
                  padb(p['g1'].reshape(1, -1)), padb(p['s1'].reshape(1, -1)),
                  pad8(p['W2']), padb(p['b2'].reshape(1, -1))]
    for nm in ('pred1', 'pred2', 'pred3'):
        wlist += [pad8(params[nm]['W']), padb(params[nm]['b'].reshape(1, -1))]

    args = [stu_rows, ex_rows,
            jnp.pad(sensitive.reshape(B, 1), ((0, 0), (0, 7))), kp,
            cls_labels.astype(f32).T, mean_stu.reshape(1, -1)] + wlist
    out, rev, cls = pl.pallas_call(
        _dense_kernel,
        out_shape=[
            jax.ShapeDtypeStruct((B, 1), f32),
            jax.ShapeDtypeStruct((1, 1), f32),
            jax.ShapeDtypeStruct((1, 1), f32),
        ],
        compiler_params=pltpu.CompilerParams(
            vmem_limit_bytes=63 * 1024 * 1024),
        name="dense_head",
    )(*args)
    return out, rev[0, 0], cls[0, 0]


def kernel(stu_id, input_exercise, input_knowledge_point, sensitive, labels,
           cls_labels, edge_stu, edge_exer, params):
    S, K = params['student_emb'].shape
    E = params['k_diff'].shape[0]
    NE = edge_stu.shape[0]
    B = stu_id.shape[0]
    f32 = jnp.float32

    ones = jnp.ones((NE,), f32)
    d_u = jax.ops.segment_sum(ones, edge_stu, num_segments=S)
    d_i = jax.ops.segment_sum(ones, edge_exer, num_segments=E)

    emb, mean_stu, exer_emb = _graph_embeddings(
        edge_stu, edge_exer, params['student_emb'], params['k_diff'],
        params['alpha'], params['beta'], d_u, d_i)

    exer_pack = jax.lax.dynamic_update_slice(
        exer_emb, params['e_diff'], (0, K))

    stu_rows = emb[stu_id]
    ex_rows = exer_pack[input_exercise]

    output, reverse_loss, cls_loss = _dense_head(
        stu_rows, ex_rows, sensitive, input_knowledge_point, cls_labels,
        mean_stu, params)
    return output, reverse_loss, cls_loss


# speedup vs baseline: 1.3258x; 1.0409x over previous
import functools

import jax
import jax.numpy as jnp
from jax.experimental import pallas as pl
from jax.experimental.pallas import tpu as pltpu

EPS = 1e-5
SENS_MEAN = 0.00014507418272432547
LANES = 128


def _cdiv(a, b):
    return (a + b - 1) // b


WU = 128


SUBR = 16


def _apass_acc(rp_ref, su_ref, cu_ref, tbl_ref, acc_ref, g_scr, oh_scr):
    b = pl.program_id(0)
    lo = rp_ref[b]
    hi = rp_ref[b + 1]
    acc_ref[...] = jnp.zeros_like(acc_ref)
    a0 = lo // LANES
    nch = (hi - a0 * LANES + SUBR * LANES - 1) // (SUBR * LANES)

    iota_sub = jax.lax.broadcasted_iota(jnp.int32, (WU, LANES), 0)
    iota_lane = jax.lax.broadcasted_iota(jnp.int32, (1, LANES), 1)

    def body(ch, _):
        r0 = a0 + ch * SUBR
        for sm in range(SUBR):
            r = r0 + sm
            suv = su_ref[r, 0, :].reshape(1, LANES)
            pos = r * LANES + iota_lane
            valid = (pos >= lo) & (pos < hi)
            local = suv - b * WU
            oh_scr[:, sm * LANES:(sm + 1) * LANES] = jnp.where(
                (iota_sub == local) & valid, 1.0, 0.0)
            for mi in range(LANES):
                idx = cu_ref[r, 0, mi]
                g_scr[sm * LANES + mi, :] = tbl_ref[idx, 0, :]
        acc_ref[...] += jnp.dot(oh_scr[...], g_scr[...],
                                preferred_element_type=jnp.float32)
        return 0

    jax.lax.fori_loop(0, nch, body, 0)


def _a12_kernel(rp_ref, su_ref, cu_ref, tbl_ref, sp_ref,
                u2_ref, s01_ref, ut2_ref, acc_ref, g_scr, oh_scr):
    _apass_acc(rp_ref, su_ref, cu_ref, tbl_ref, acc_ref, g_scr, oh_scr)
    acc = acc_ref[...]
    a_e0 = acc[:, 0:LANES]
    a_i1 = acc[:, LANES:2 * LANES]
    sp = sp_ref[...]
    du = sp[:, 68:69]
    sqdu = sp[:, 69:70]
    u1 = sqdu * a_e0 + du * sp
    u2 = sqdu * a_i1 + du * u1
    lane = jax.lax.broadcasted_iota(jnp.int32, (WU, LANES), 1)
    feat = lane < 68
    u2_ref[...] = u2
    s01_ref[...] = jnp.where(feat, sp + u1 + u2, sp)
    ut2_ref[...] = jnp.where(feat, sqdu * u2, 0.0)


def _a34_kernel(rp_ref, su_ref, cu_ref, tbl_ref, s01_ref, u2_ref,
                emb_ref, ut3_ref, msum_ref, acc_ref, g_scr, oh_scr):
    _apass_acc(rp_ref, su_ref, cu_ref, tbl_ref, acc_ref, g_scr, oh_scr)
    acc = acc_ref[...]
    a_i2 = acc[:, 0:LANES]
    a_i3 = acc[:, LANES:2 * LANES]
    s01 = s01_ref[...]
    u2 = u2_ref[...]
    du = s01[:, 68:69]
    sqdu = s01[:, 69:70]
    u3 = sqdu * a_i2 + du * u2
    u4 = sqdu * a_i3 + du * u3
    lane = jax.lax.broadcasted_iota(jnp.int32, (WU, LANES), 1)
    feat = lane < 68
    emb = jnp.where(feat, s01 + u3 + u4, s01)
    emb_ref[...] = emb
    ut3_ref[...] = jnp.where(feat, sqdu * u3, 0.0)
    msum_ref[...] = jnp.sum(jnp.where(feat, emb, 0.0), axis=0).reshape(1, 1, LANES)


def _tpass_kernel(rp_ref, ki_ref, si_ref, stbl_ref, out_ref, acc_ref, g_scr,
                  oh_scr, *, ep, nbq, qsize, we):
    q = pl.program_id(0)
    wb = pl.program_id(1)
    blk = q * nbq + wb
    lo = rp_ref[blk]
    hi = rp_ref[blk + 1]
    acc_ref[...] = jnp.zeros_like(acc_ref)
    a0 = lo // LANES
    nch = (hi - a0 * LANES + SUBR * LANES - 1) // (SUBR * LANES)

    iota_sub = jax.lax.broadcasted_iota(jnp.int32, (we, LANES), 0)
    iota_lane = jax.lax.broadcasted_iota(jnp.int32, (1, LANES), 1)
    base = q * ep + wb * we
    qoff = q * qsize

    def body(ch, _):
        r0 = a0 + ch * SUBR
        for sm in range(SUBR):
            r = r0 + sm
            kiv = ki_ref[r, 0, :].reshape(1, LANES)
            pos = r * LANES + iota_lane
            valid = (pos >= lo) & (pos < hi)
            local = kiv - base
            oh_scr[:, sm * LANES:(sm + 1) * LANES] = jnp.where(
                (iota_sub == local) & valid, 1.0, 0.0)
            for mi in range(LANES):
                idx = si_ref[r, 0, mi] - qoff
                idx = jnp.clip(idx, 0, qsize - 1)
                g_scr[sm * LANES + mi, :] = stbl_ref[idx, 0, :]
        acc_ref[...] += jnp.dot(oh_scr[...], g_scr[...],
                                preferred_element_type=jnp.float32)
        return 0

    jax.lax.fori_loop(0, nch, body, 0)
    out_ref[...] = acc_ref[...].reshape(1, we, LANES)


def _graph_embeddings(edge_stu, edge_exer, s0, e0, alpha, beta, d_u, d_i):
    S, K = s0.shape
    E = e0.shape[0]
    NE = edge_stu.shape[0]

    NBU = _cdiv(S, 4 * WU) * 4
    SP = NBU * WU
    QS = SP // 4
    WE = 64
    NBQ = _cdiv(E, WE)
    EP = NBQ * WE
    NBI = 4 * NBQ
    NR = _cdiv(NE, LANES) + 2 * SUBR
    NEP = NR * LANES

    f32 = jnp.float32
    du = 1.0 / (d_u + 1.0)
    di = 1.0 / (d_i + 1.0)
    sqdu = jnp.sqrt(du)
    sqdi = jnp.sqrt(di)

    su, cu = jax.lax.sort((edge_stu, edge_exer), num_keys=1)
    qid = edge_stu // QS
    kI, sI = jax.lax.sort((qid * EP + edge_exer, edge_stu), num_keys=1)

    def pad_edges(a, sent):
        a = jnp.concatenate([a, jnp.full((NEP - NE,), sent, a.dtype)])
        return a.reshape(NR, 1, LANES)

    su3 = pad_edges(su, S)
    cu3 = pad_edges(cu, 0)
    kI3 = pad_edges(kI, 4 * EP)
    sI3 = pad_edges(sI, 0)

    degu_p = jnp.concatenate([d_u, jnp.zeros((SP - S,), f32)])
    cnt_u = degu_p.reshape(NBU, WU).sum(1)
    rpU = jnp.concatenate([jnp.zeros((1,), f32), jnp.cumsum(cnt_u)]).astype(jnp.int32)
    degq = jax.ops.segment_sum(jnp.ones((NE,), f32), qid * EP + edge_exer,
                               num_segments=4 * EP)
    cnt_i = degq.reshape(NBI, WE).sum(1)
    rpI = jnp.concatenate([jnp.zeros((1,), f32), jnp.cumsum(cnt_i)]).astype(jnp.int32)

    def padlane(x, rows, width=LANES):
        out = jnp.zeros((rows, width), f32)
        return out.at[:x.shape[0], :x.shape[1]].set(x)

    sp_pack = padlane(
        jnp.concatenate([s0, du[:, None], sqdu[:, None], alpha, beta], axis=1), SP)
    st0 = padlane(s0 * sqdu[:, None], SP).reshape(SP, 1, LANES)

    e0p = padlane(e0, EP)
    sqdi_p = jnp.concatenate([sqdi, jnp.zeros((EP - E,), f32)])[:, None]
    di_p = jnp.concatenate([di, jnp.zeros((EP - E,), f32)])[:, None]

    tp = functools.partial(_tpass_kernel, ep=EP, nbq=NBQ, qsize=QS, we=WE)
    tpass = pl.pallas_call(
        tp,
        grid_spec=pltpu.PrefetchScalarGridSpec(
            num_scalar_prefetch=1,
            grid=(4, NBQ),
            in_specs=[
                pl.BlockSpec((NR, 1, LANES), lambda q, w, rp: (0, 0, 0)),
                pl.BlockSpec((NR, 1, LANES), lambda q, w, rp: (0, 0, 0)),
                pl.BlockSpec((QS, 1, LANES), lambda q, w, rp: (q, 0, 0)),
            ],
            out_specs=pl.BlockSpec((1, WE, LANES), lambda q, w, rp: (q * NBQ + w, 0, 0)),
            scratch_shapes=[
                pltpu.VMEM((WE, LANES), f32),
                pltpu.VMEM((SUBR * LANES, LANES), f32),
                pltpu.VMEM((WE, SUBR * LANES), f32),
            ],
        ),
        out_shape=jax.ShapeDtypeStruct((NBI, WE, LANES), f32),
        compiler_params=pltpu.CompilerParams(
            dimension_semantics=("parallel", "arbitrary")),
        name="tpass",
    )

    def run_t(stbl):
        acc = tpass(rpI, kI3, sI3, stbl)
        return acc.reshape(4, NBQ * WE, LANES).sum(0)

    bt_s0 = run_t(st0)
    i1 = sqdi_p * bt_s0 + di_p * e0p

    tblA12 = jnp.concatenate([sqdi_p * e0p, sqdi_p * i1], axis=1).reshape(EP, 1, 2 * LANES)

    apass12 = pl.pallas_call(
        _a12_kernel,
        grid_spec=pltpu.PrefetchScalarGridSpec(
            num_scalar_prefetch=1,
            grid=(NBU,),
            in_specs=[
                pl.BlockSpec((NR, 1, LANES), lambda b, rp: (0, 0, 0)),
                pl.BlockSpec((NR, 1, LANES), lambda b, rp: (0, 0, 0)),
                pl.BlockSpec((EP, 1, 2 * LANES), lambda b, rp: (0, 0, 0)),
                pl.BlockSpec((WU, LANES), lambda b, rp: (b, 0)),
            ],
            out_specs=[
                pl.BlockSpec((WU, LANES), lambda b, rp: (b, 0)),
                pl.BlockSpec((WU, LANES), lambda b, rp: (b, 0)),
                pl.BlockSpec((WU, LANES), lambda b, rp: (b, 0)),
            ],
            scratch_shapes=[
                pltpu.VMEM((WU, 2 * LANES), f32),
                pltpu.VMEM((SUBR * LANES, 2 * LANES), f32),
                pltpu.VMEM((WU, SUBR * LANES), f32),
            ],
        ),
        out_shape=[
            jax.ShapeDtypeStruct((SP, LANES), f32),
            jax.ShapeDtypeStruct((SP, LANES), f32),
            jax.ShapeDtypeStruct((SP, LANES), f32),
        ],
        compiler_params=pltpu.CompilerParams(dimension_semantics=("parallel",)),
        name="apass12",
    )
    u2a, s01a, ut2a = apass12(rpU, su3, cu3, tblA12, sp_pack)

    bt_u2 = run_t(ut2a.reshape(SP, 1, LANES))
    i2 = sqdi_p * bt_u2 + di_p * i1
    i3 = i2 + di_p * (i2 - i1)

    tblA34 = jnp.concatenate([sqdi_p * i2, sqdi_p * i3], axis=1).reshape(EP, 1, 2 * LANES)

    apass34 = pl.pallas_call(
        _a34_kernel,
        grid_spec=pltpu.PrefetchScalarGridSpec(
            num_scalar_prefetch=1,
            grid=(NBU,),
            in_specs=[
                pl.BlockSpec((NR, 1, LANES), lambda b, rp: (0, 0, 0)),
                pl.BlockSpec((NR, 1, LANES), lambda b, rp: (0, 0, 0)),
                pl.BlockSpec((EP, 1, 2 * LANES), lambda b, rp: (0, 0, 0)),
                pl.BlockSpec((WU, LANES), lambda b, rp: (b, 0)),
                pl.BlockSpec((WU, LANES), lambda b, rp: (b, 0)),
            ],
            out_specs=[
                pl.BlockSpec((WU, LANES), lambda b, rp: (b, 0)),
                pl.BlockSpec((WU, LANES), lambda b, rp: (b, 0)),
                pl.BlockSpec((1, 1, LANES), lambda b, rp: (b, 0, 0)),
            ],
            scratch_shapes=[
                pltpu.VMEM((WU, 2 * LANES), f32),
                pltpu.VMEM((SUBR * LANES, 2 * LANES), f32),
                pltpu.VMEM((WU, SUBR * LANES), f32),
            ],
        ),
        out_shape=[
            jax.ShapeDtypeStruct((SP, LANES), f32),
            jax.ShapeDtypeStruct((SP, LANES), f32),
            jax.ShapeDtypeStruct((NBU, 1, LANES), f32),
        ],
        compiler_params=pltpu.CompilerParams(dimension_semantics=("parallel",)),
        name="apass34",
    )
    emb, ut3a, msum = apass34(rpU, su3, cu3, tblA34, s01a, u2a)

    bt_u3 = run_t(ut3a.reshape(SP, 1, LANES))
    i4 = sqdi_p * bt_u3 + di_p * i3

    exer_emb = e0p + i1 + i2 + i3 + i4
    mean_stu = msum.sum(axis=(0, 1))[:K] / S
    return emb, mean_stu, exer_emb


def _matT(x, w):
    return jax.lax.dot_general(x, w, (((1,), (1,)), ((), ())),
                               preferred_element_type=jnp.float32)


def _lin_r(x, wr, br):
    return _matT(x, wr[...]) + br[...]


def _bn_r(x, gr, sr):
    m = jnp.mean(x, axis=0, keepdims=True)
    v = jnp.mean(jnp.square(x - m), axis=0, keepdims=True)
    return gr[...] * (x - m) * jax.lax.rsqrt(v + EPS) + sr[...]


def _mlp3_r(x, p):
    h = jax.nn.relu(_bn_r(_lin_r(x, p[0], p[1]), p[2], p[3]))
    h = jax.nn.relu(_bn_r(_lin_r(h, p[4], p[5]), p[6], p[7]))
    return _bn_r(_lin_r(h, p[8], p[9]), p[10], p[11])


def _rev_r(x, p):
    h = jax.nn.relu(_bn_r(_lin_r(x, p[0], p[1]), p[2], p[3]))
    return _bn_r(_lin_r(h, p[4], p[5]), p[6], p[7])[:, 0:1]


def _pos_lin_r(x, wr, br):
    w = wr[...]
    return _matT(x, 2.0 * jax.nn.relu(-w) + w) + br[...]


def _dense_kernel(stu_ref, ex_ref, sens_ref, kp_ref, clsl_ref, mean_ref,
                  *refs):
    out_ref, rev_ref, cls_ref = refs[-3:]
    w = list(refs[:-3])
    K = 68
    B = stu_ref.shape[0]
    p_comb, w = w[:12], w[12:]
    p_sens, w = w[:12], w[12:]
    p_sdense, w = w[:12], w[12:]
    p_rev, w = w[:8], w[8:]
    p_cls = [None] * 5
    for i in range(5):
        p_cls[i], w = w[:6], w[6:]
    (pred1w, pred1b, pred2w, pred2b, pred3w, pred3b) = w

    stu = stu_ref[...]
    stu_feat = stu[:, :K]
    alpha = jax.nn.sigmoid(stu[:, 70:71])
    beta = jax.nn.sigmoid(stu[:, 71:72])
    ex = ex_ref[...]
    k_diff = jax.nn.sigmoid(ex[:, :K])
    e_diff = jax.nn.sigmoid(ex[:, K:K + 1])
    sens8 = sens_ref[...]
    sens = sens8[:, 0:1]

    kp = kp_ref[...]
    iota_k = jax.lax.broadcasted_iota(jnp.int32, (B, K), 1)
    mh = jnp.zeros((B, K), jnp.float32)
    for j in range(4):
        mh = jnp.maximum(mh, jnp.where(iota_k == kp[:, j:j + 1], 1.0, 0.0))

    sens_feat = _mlp3_r(sens8, p_sens)
    Uf_features = _mlp3_r(jnp.concatenate([stu_feat, sens_feat], axis=-1),
                          p_comb)
    Ud_features = _mlp3_r(sens_feat, p_sdense)
    Uf_rev = _rev_r(Uf_features, p_rev)
    Ud_rev = _rev_r(Ud_features, p_rev)

    reverse_loss = (jnp.mean(jnp.square(Uf_rev - SENS_MEAN))
                    + jnp.mean(jnp.square(Ud_rev - sens)))
    rev_ref[...] = reverse_loss.reshape(1, 1)

    Uf = jax.nn.sigmoid(Uf_features)
    Ud = jax.nn.sigmoid(Ud_features)
    stat_emb = jax.nn.sigmoid((1.0 - alpha) * Uf + alpha * Ud)

    con_stu = jnp.broadcast_to(mean_ref[...], (B, K))
    con_col = jnp.full((B, 8), SENS_MEAN, jnp.float32)
    con_sens = _mlp3_r(con_col, p_sens)
    con_Uf = jax.nn.sigmoid(
        _mlp3_r(jnp.concatenate([con_stu, con_sens], axis=-1), p_comb))
    con_stat = jax.nn.sigmoid((1.0 - alpha) * con_Uf + alpha * Ud)

    clsl = clsl_ref[...]
    cls_loss = 0.0
    for i in range(5):
        pc = p_cls[i]
        h = jax.nn.relu(_bn_r(_lin_r(Uf_features, pc[0], pc[1]), pc[2], pc[3]))
        oi = jax.nn.sigmoid(_lin_r(h, pc[4], pc[5])[:, 0:1])
        y = clsl[:, i:i + 1]
        cls_loss = cls_loss + jnp.mean(jax.nn.softplus(oi) - oi * y)
    cls_ref[...] = (cls_loss / 5.0).reshape(1, 1)

    debias_theta = jax.nn.sigmoid(stat_emb - beta * con_stat)
    x = e_diff * (debias_theta - k_diff) * mh
    x = jax.nn.sigmoid(_pos_lin_r(x, pred1w, pred1b))
    x = jax.nn.sigmoid(_pos_lin_r(x, pred2w, pred2b))
    out_ref[...] = jax.nn.sigmoid(_pos_lin_r(x, pred3w, pred3b)[:, 0:1])


def _dense_head(stu_rows, ex_rows, sensitive, kp, cls_labels, mean_stu, params):
    B = stu_rows.shape[0]
    f32 = jnp.float32

    def pad8(a):
        if a.shape[0] == 1 and a.shape[-1] > 1:
            a = jnp.pad(a, ((0, 7), (0, 0)))
        if a.shape[-1] == 1 and a.shape[0] > 1:
            a = jnp.pad(a, ((0, 0), (0, 7)))
        return a

    def padb(a):
        if a.shape[-1] == 1:
            return jnp.pad(a, ((0, 0), (0, 7)))
        return a

    def blk(p, n):
        out = []
        for i in range(n):
            j = i + 1
            out += [pad8(p['W%d' % j]), padb(p['b%d' % j].reshape(1, -1)),
                    padb(p['g%d' % j].reshape(1, -1)),
                    padb(p['s%d' % j].reshape(1, -1))]
        return out

    wlist = blk(params['combine'], 3) + blk(params['sens'], 3) \
        + blk(params['sens_dense'], 3) + blk(params['sens_rev'], 2)
    for i in range(5):
        p = params['cls'][i]
        wlist += [pad8(p['W1']), padb(p['b1'].reshape(1, -1)),
                  padb(p['g1'].reshape(1, -1)), padb(p['s1'].reshape(1, -1)),
                  pad8(p['W2']), padb(p['b2'].reshape(1, -1))]
    for nm in ('pred1', 'pred2', 'pred3'):
        wlist += [pad8(params[nm]['W']), padb(params[nm]['b'].reshape(1, -1))]

    args = [stu_rows, ex_rows,
            jnp.pad(sensitive.reshape(B, 1), ((0, 0), (0, 7))), kp,
            cls_labels.astype(f32).T, mean_stu.reshape(1, -1)] + wlist
    out, rev, cls = pl.pallas_call(
        _dense_kernel,
        out_shape=[
            jax.ShapeDtypeStruct((B, 1), f32),
            jax.ShapeDtypeStruct((1, 1), f32),
            jax.ShapeDtypeStruct((1, 1), f32),
        ],
        compiler_params=pltpu.CompilerParams(
            vmem_limit_bytes=63 * 1024 * 1024),
        name="dense_head",
    )(*args)
    return out, rev[0, 0], cls[0, 0]


def kernel(stu_id, input_exercise, input_knowledge_point, sensitive, labels,
           cls_labels, edge_stu, edge_exer, params):
    S, K = params['student_emb'].shape
    E = params['k_diff'].shape[0]
    NE = edge_stu.shape[0]
    B = stu_id.shape[0]
    f32 = jnp.float32

    ones = jnp.ones((NE,), f32)
    d_u = jax.ops.segment_sum(ones, edge_stu, num_segments=S)
    d_i = jax.ops.segment_sum(ones, edge_exer, num_segments=E)

    emb, mean_stu, exer_emb = _graph_embeddings(
        edge_stu, edge_exer, params['student_emb'], params['k_diff'],
        params['alpha'], params['beta'], d_u, d_i)

    exer_pack = jax.lax.dynamic_update_slice(
        exer_emb, params['e_diff'], (0, K))

    stu_rows = emb[stu_id]
    ex_rows = exer_pack[input_exercise]

    output, reverse_loss, cls_loss = _dense_head(
        stu_rows, ex_rows, sensitive, input_knowledge_point, cls_labels,
        mean_stu, params)
    return output, reverse_loss, cls_loss
